# Initial kernel scaffold; baseline (speedup 1.0000x reference)
import jax
import jax.numpy as jnp
from jax.experimental import pallas as pl


def kernel(scores, prev_scores):
    raise NotImplementedError("write your pallas kernel here")


# speedup vs baseline: 57.2442x; 57.2442x over previous
import functools
import jax
import jax.numpy as jnp
from jax.experimental import pallas as pl
from jax.experimental.pallas import tpu as pltpu

_BEAM = 16
_PRE_BEAM = 24
_VOCAB = 1_000_000
_NBLK = 1000
_BLK = 1000
_IBIG = 2 ** 30


def _beam_kernel(x_ref, prev_ref, vals_ref, beams_ref, toks_ref,
                 cvals_s, cids_s, lse_s):
    b = pl.program_id(0)
    x = x_ref[0]

    bm = jnp.max(x, axis=1, keepdims=True)
    m = jnp.max(bm)
    s = jnp.sum(jnp.exp(x - m))
    lse = m + jnp.log(s)
    lse_s[pl.ds(b, 1), :] = jnp.full((1, 1), 0.0, jnp.float32) + lse

    row_iota = jax.lax.broadcasted_iota(jnp.int32, (_NBLK, 1), 0)
    col_iota = jax.lax.broadcasted_iota(jnp.int32, (1, _BLK), 1)
    i24 = jax.lax.broadcasted_iota(jnp.int32, (1, _PRE_BEAM), 1)

    def body(i, carry):
        bm, vals, ids = carry
        vmax = jnp.max(bm)
        rix = jnp.min(jnp.where(bm == vmax, row_iota, _IBIG))
        row = x_ref[0, pl.ds(rix, 1), :]
        cix = jnp.min(jnp.where(row == vmax, col_iota, _IBIG))
        new_row = jnp.where(col_iota == cix, -jnp.inf, row)
        x_ref[0, pl.ds(rix, 1), :] = new_row
        bm = jnp.where(row_iota == rix, jnp.max(new_row), bm)
        vals = jnp.where(i24 == i, vmax, vals)
        ids = jnp.where(i24 == i, rix * _BLK + cix, ids)
        return bm, vals, ids

    vals0 = jnp.full((1, _PRE_BEAM), -jnp.inf, jnp.float32)
    ids0 = jnp.zeros((1, _PRE_BEAM), jnp.int32)
    _, vals, ids = jax.lax.fori_loop(0, _PRE_BEAM, body, (bm, vals0, ids0))

    cvals_s[pl.ds(b, 1), :] = vals
    cids_s[pl.ds(b, 1), :] = ids

    @pl.when(b == _BEAM - 1)
    def _():
        total = cvals_s[...] - lse_s[...] + prev_ref[...]
        beam_iota = jax.lax.broadcasted_iota(jnp.int32, (_BEAM, _PRE_BEAM), 0)
        flat = beam_iota * _VOCAB + cids_s[...]
        lane16 = jax.lax.broadcasted_iota(jnp.int32, (1, _BEAM), 1)

        def fbody(i, carry):
            total, ovals, oflat = carry
            vmax = jnp.max(total)
            fi = jnp.min(jnp.where(total == vmax, flat, _IBIG))
            ovals = jnp.where(lane16 == i, vmax, ovals)
            oflat = jnp.where(lane16 == i, fi, oflat)
            total = jnp.where(flat == fi, -jnp.inf, total)
            return total, ovals, oflat

        ovals0 = jnp.zeros((1, _BEAM), jnp.float32)
        oflat0 = jnp.zeros((1, _BEAM), jnp.int32)
        _, ovals, oflat = jax.lax.fori_loop(
            0, _BEAM, fbody, (total, ovals0, oflat0))

        vals_ref[...] = ovals
        beams_ref[...] = oflat // _VOCAB
        toks_ref[...] = oflat - (oflat // _VOCAB) * _VOCAB


@jax.jit
def kernel(scores, prev_scores):
    x = scores.reshape(_BEAM, _NBLK, _BLK)
    prev = prev_scores.reshape(_BEAM, 1)

    grid = (_BEAM,)
    out = pl.pallas_call(
        _beam_kernel,
        grid=grid,
        in_specs=[
            pl.BlockSpec((1, _NBLK, _BLK), lambda b: (b, 0, 0)),
            pl.BlockSpec((_BEAM, 1), lambda b: (0, 0)),
        ],
        out_specs=[
            pl.BlockSpec((1, _BEAM), lambda b: (0, 0)),
            pl.BlockSpec((1, _BEAM), lambda b: (0, 0)),
            pl.BlockSpec((1, _BEAM), lambda b: (0, 0)),
        ],
        out_shape=[
            jax.ShapeDtypeStruct((1, _BEAM), jnp.float32),
            jax.ShapeDtypeStruct((1, _BEAM), jnp.int32),
            jax.ShapeDtypeStruct((1, _BEAM), jnp.int32),
        ],
        scratch_shapes=[
            pltpu.VMEM((_BEAM, _PRE_BEAM), jnp.float32),
            pltpu.VMEM((_BEAM, _PRE_BEAM), jnp.int32),
            pltpu.VMEM((_BEAM, 1), jnp.float32),
        ],
    )(x, prev)

    top_vals, beam_ids, token_ids = out
    return top_vals.reshape(_BEAM), beam_ids.reshape(_BEAM), token_ids.reshape(_BEAM)
